# baseline (speedup 1.0000x reference)
import functools

import numpy as np
import jax
import jax.numpy as jnp
from jax import lax
from jax.experimental import pallas as pl
from jax.experimental.pallas import tpu as pltpu
from jax.experimental.pallas import tpu_sc as plsc

T = 8
NUM_TOKENS = 2048
EMBED_DIM = 128
NUM_NODES = 50000
COMP_LEN = 8
COMP_DIM = 32
NN = 8192
MAX_LEN = 1024
POOL = EMBED_DIM // COMP_DIM
P_TOT = T * NUM_TOKENS

CH = 128
N_CH = NN // CH
H_CH = MAX_LEN // CH

NQ = 4
CPC = COMP_LEN // 2
NROW = 51200
ZCH = NROW // NQ
ACH = P_TOT // NQ
JCH = N_CH // NQ
HCH_T = H_CH // NQ

_ZEROS = np.zeros((ZCH,), np.float32)
_ONES = np.ones((CH,), np.float32)
_PMAT = np.repeat(np.eye(COMP_DIM, dtype=np.float32), POOL, axis=0) / POOL


def _sc_weights(si, nid, pos):
    mesh = plsc.VectorSubcoreMesh(core_axis_name="c", subcore_axis_name="s")

    @functools.partial(
        pl.kernel,
        out_type=jax.ShapeDtypeStruct((COMP_LEN, P_TOT), jnp.float32),
        mesh=mesh,
        scratch_types=[
            pltpu.VMEM_SHARED((CPC * NROW,), jnp.float32),
            pltpu.VMEM_SHARED((CPC * P_TOT,), jnp.float32),
            pltpu.VMEM((HCH_T, CH), jnp.int32),
            pltpu.VMEM((JCH * CH,), jnp.int32),
            pltpu.VMEM((JCH, CH), jnp.int32),
            pltpu.VMEM((JCH, CH), jnp.float32),
            pltpu.VMEM((CH,), jnp.float32),
            pltpu.VMEM((ZCH,), jnp.float32),
            pltpu.VMEM((ACH,), jnp.float32),
            pltpu.SemaphoreType.DMA,
        ],
    )
    def k(si_hbm, nid_hbm, pos_hbm, zeros_hbm, ones_hbm, w_hbm,
          counts_s, acc_s, sif_v, nid_v, posf_v, wt_v, ones_v,
          zeros_v, stage_v, sem):
        cid = lax.axis_index("c")
        sid = lax.axis_index("s")
        lc = sid % NQ
        comp = cid + 2 * lc
        q = sid // NQ

        cnt_base = pl.multiple_of(lc * NROW + q * ZCH, 8)
        acc_base = pl.multiple_of(lc * P_TOT + q * ACH, 8)
        my_counts = counts_s.at[pl.ds(pl.multiple_of(lc * NROW, 8), NROW)]
        my_acc = acc_s.at[pl.ds(pl.multiple_of(lc * P_TOT, 8), P_TOT)]

        pend = [
            pltpu.async_copy(zeros_hbm, zeros_v, sem),
            pltpu.async_copy(ones_hbm, ones_v, sem),
            pltpu.async_copy(
                nid_hbm.at[pl.ds(q * (JCH * CH), JCH * CH)], nid_v, sem),
            pltpu.async_copy(si_hbm.at[comp].at[pl.ds(q * HCH_T, HCH_T)],
                             sif_v, sem),
            pltpu.async_copy(pos_hbm.at[pl.ds(q * JCH, JCH)], posf_v, sem),
        ]
        for d in pend:
            d.wait()
        pltpu.sync_copy(zeros_v, counts_s.at[pl.ds(cnt_base, ZCH)])
        plsc.subcore_barrier()

        pend = [
            pltpu.async_copy(ones_v, my_counts.at[sif_v.at[i]], sem, add=True)
            for i in range(HCH_T)
        ]
        pend.append(
            pltpu.async_copy(zeros_v.at[pl.ds(0, ACH)],
                             acc_s.at[pl.ds(acc_base, ACH)], sem))
        for d in pend:
            d.wait()
        plsc.subcore_barrier()

        pend = [
            pltpu.async_copy(my_counts.at[nid_v.at[pl.ds(i * CH, CH)]],
                             wt_v.at[i], sem)
            for i in range(JCH)
        ]
        for d in pend:
            d.wait()
        pend = [
            pltpu.async_copy(wt_v.at[i], my_acc.at[posf_v.at[i]], sem, add=True)
            for i in range(JCH)
        ]
        for d in pend:
            d.wait()
        plsc.subcore_barrier()

        pltpu.sync_copy(acc_s.at[pl.ds(acc_base, ACH)], stage_v)
        pltpu.sync_copy(stage_v, w_hbm.at[comp].at[pl.ds(q * ACH, ACH)])

    return k(si.reshape(COMP_LEN, H_CH, CH), nid, pos.reshape(N_CH, CH),
             jnp.asarray(_ZEROS), jnp.asarray(_ONES))


def _tc_finish(x, w, g1, b1, pmat, g2, b2, interpret=False):
    TB = 8

    def body(x_ref, w_ref, g1_ref, b1_ref, pm_ref, g2_ref, b2_ref, o_ref):
        for k in range(TB):
            xt = x_ref[k]
            mu = jnp.mean(xt, axis=-1, keepdims=True)
            var = jnp.mean(jnp.square(xt - mu), axis=-1, keepdims=True)
            xn = (xt - mu) * lax.rsqrt(var + 1e-5) * g1_ref[0] + b1_ref[0]
            acc = jnp.dot(w_ref[:, k * NUM_TOKENS:(k + 1) * NUM_TOKENS], xn,
                          preferred_element_type=jnp.float32)
            comp = jnp.dot(acc, pm_ref[...],
                           preferred_element_type=jnp.float32) * (1.0 / MAX_LEN)
            m2 = jnp.mean(comp)
            v2 = jnp.mean(jnp.square(comp - m2))
            o_ref[k] = ((comp - m2) * lax.rsqrt(v2 + 1e-5) * g2_ref[...]
                        + b2_ref[...])

    return pl.pallas_call(
        body,
        grid=(T // TB,),
        in_specs=[
            pl.BlockSpec((TB, NUM_TOKENS, EMBED_DIM), lambda t: (t, 0, 0)),
            pl.BlockSpec((COMP_LEN, TB * NUM_TOKENS), lambda t: (0, t)),
            pl.BlockSpec((1, EMBED_DIM), lambda t: (0, 0)),
            pl.BlockSpec((1, EMBED_DIM), lambda t: (0, 0)),
            pl.BlockSpec((EMBED_DIM, COMP_DIM), lambda t: (0, 0)),
            pl.BlockSpec((COMP_LEN, COMP_DIM), lambda t: (0, 0)),
            pl.BlockSpec((COMP_LEN, COMP_DIM), lambda t: (0, 0)),
        ],
        out_specs=pl.BlockSpec((TB, COMP_LEN, COMP_DIM), lambda t: (t, 0, 0)),
        out_shape=jax.ShapeDtypeStruct((T, COMP_LEN, COMP_DIM), jnp.float32),
        interpret=interpret,
    )(x, w, g1, b1, pmat, g2, b2)


def kernel(x, ln1_scale, ln1_bias, ln2_scale, ln2_bias,
           node_token_pos, node_global_idx, stacked_indices):
    si = stacked_indices.astype(jnp.int32)
    nid = node_global_idx.astype(jnp.int32)
    pos = node_token_pos.astype(jnp.int32)

    w = _sc_weights(si, nid, pos)

    out = _tc_finish(x, w,
                     ln1_scale.reshape(1, EMBED_DIM),
                     ln1_bias.reshape(1, EMBED_DIM),
                     jnp.asarray(_PMAT),
                     ln2_scale.reshape(COMP_LEN, COMP_DIM),
                     ln2_bias.reshape(COMP_LEN, COMP_DIM))
    return out.reshape(T, 1, COMP_LEN * COMP_DIM)


# speedup vs baseline: 1.0620x; 1.0620x over previous
import functools

import numpy as np
import jax
import jax.numpy as jnp
from jax import lax
from jax.experimental import pallas as pl
from jax.experimental.pallas import tpu as pltpu
from jax.experimental.pallas import tpu_sc as plsc

T = 8
NUM_TOKENS = 2048
EMBED_DIM = 128
NUM_NODES = 50000
COMP_LEN = 8
COMP_DIM = 32
NN = 8192
MAX_LEN = 1024
POOL = EMBED_DIM // COMP_DIM
P_TOT = T * NUM_TOKENS

CH = 128
N_CH = NN // CH
H_CH = MAX_LEN // CH

NQ = 4
CPC = COMP_LEN // 2
NROW = 51200
ZCH = NROW // NQ
ACH = P_TOT // NQ
JCH = N_CH // NQ
HCH_T = H_CH // NQ

_ZEROS = np.zeros((ZCH,), np.float32)
_ONES = np.ones((CH,), np.float32)
_PMAT = np.repeat(np.eye(COMP_DIM, dtype=np.float32), POOL, axis=0) / POOL


def _sc_weights(si, nid, pos):
    mesh = plsc.VectorSubcoreMesh(core_axis_name="c", subcore_axis_name="s")

    @functools.partial(
        pl.kernel,
        out_type=jax.ShapeDtypeStruct((COMP_LEN, P_TOT), jnp.float32),
        mesh=mesh,
        scratch_types=[
            pltpu.VMEM_SHARED((CPC * NROW,), jnp.float32),
            pltpu.VMEM_SHARED((CPC * P_TOT,), jnp.float32),
            pltpu.VMEM((HCH_T, CH), jnp.int32),
            pltpu.VMEM((JCH * CH,), jnp.int32),
            pltpu.VMEM((JCH, CH), jnp.int32),
            pltpu.VMEM((JCH, CH), jnp.float32),
            pltpu.VMEM((CH,), jnp.float32),
            pltpu.VMEM((ZCH,), jnp.float32),
            pltpu.VMEM((ACH,), jnp.float32),
            pltpu.SemaphoreType.DMA,
        ],
    )
    def k(si_hbm, nid_hbm, pos_hbm, zeros_hbm, ones_hbm, w_hbm,
          counts_s, acc_s, sif_v, nid_v, posf_v, wt_v, ones_v,
          zeros_v, stage_v, sem):
        cid = lax.axis_index("c")
        sid = lax.axis_index("s")
        lc = sid % NQ
        comp = cid + 2 * lc
        q = sid // NQ

        cnt_base = pl.multiple_of(lc * NROW + q * ZCH, 8)
        acc_base = pl.multiple_of(lc * P_TOT + q * ACH, 8)
        my_counts = counts_s.at[pl.ds(pl.multiple_of(lc * NROW, 8), NROW)]
        my_acc = acc_s.at[pl.ds(pl.multiple_of(lc * P_TOT, 8), P_TOT)]

        pend = [
            pltpu.async_copy(zeros_hbm, zeros_v, sem),
            pltpu.async_copy(ones_hbm, ones_v, sem),
            pltpu.async_copy(
                nid_hbm.at[pl.ds(q * (JCH * CH), JCH * CH)], nid_v, sem),
            pltpu.async_copy(si_hbm.at[comp].at[pl.ds(q * HCH_T, HCH_T)],
                             sif_v, sem),
            pltpu.async_copy(pos_hbm.at[pl.ds(q * JCH, JCH)], posf_v, sem),
        ]
        for d in pend:
            d.wait()
        pltpu.sync_copy(zeros_v, counts_s.at[pl.ds(cnt_base, ZCH)])
        plsc.subcore_barrier()

        pend = [
            pltpu.async_copy(ones_v, my_counts.at[sif_v.at[i]], sem, add=True)
            for i in range(HCH_T)
        ]
        pend.append(
            pltpu.async_copy(zeros_v.at[pl.ds(0, ACH)],
                             acc_s.at[pl.ds(acc_base, ACH)], sem))
        for d in pend:
            d.wait()
        plsc.subcore_barrier()

        pend = [
            pltpu.async_copy(my_counts.at[nid_v.at[pl.ds(i * CH, CH)]],
                             wt_v.at[i], sem)
            for i in range(JCH)
        ]
        for d in pend:
            d.wait()
        pend = [
            pltpu.async_copy(wt_v.at[i], my_acc.at[posf_v.at[i]], sem, add=True)
            for i in range(JCH)
        ]
        for d in pend:
            d.wait()
        plsc.subcore_barrier()

        pltpu.sync_copy(acc_s.at[pl.ds(acc_base, ACH)], stage_v)
        pltpu.sync_copy(stage_v, w_hbm.at[comp].at[pl.ds(q * ACH, ACH)])

    return k(si.reshape(COMP_LEN, H_CH, CH), nid, pos.reshape(N_CH, CH),
             jnp.asarray(_ZEROS), jnp.asarray(_ONES))


def _tc_finish(x, w, g1, b1, pmat, g2, b2, interpret=False):
    TB = 2

    def body(x_ref, w_ref, g1_ref, b1_ref, pm_ref, g2_ref, b2_ref, o_ref):
        for k in range(TB):
            xt = x_ref[k]
            mu = jnp.mean(xt, axis=-1, keepdims=True)
            var = jnp.mean(jnp.square(xt - mu), axis=-1, keepdims=True)
            xn = (xt - mu) * lax.rsqrt(var + 1e-5) * g1_ref[0] + b1_ref[0]
            acc = jnp.dot(w_ref[:, k * NUM_TOKENS:(k + 1) * NUM_TOKENS], xn,
                          preferred_element_type=jnp.float32)
            comp = jnp.dot(acc, pm_ref[...],
                           preferred_element_type=jnp.float32) * (1.0 / MAX_LEN)
            m2 = jnp.mean(comp)
            v2 = jnp.mean(jnp.square(comp - m2))
            o_ref[k] = ((comp - m2) * lax.rsqrt(v2 + 1e-5) * g2_ref[...]
                        + b2_ref[...])

    return pl.pallas_call(
        body,
        grid=(T // TB,),
        in_specs=[
            pl.BlockSpec((TB, NUM_TOKENS, EMBED_DIM), lambda t: (t, 0, 0)),
            pl.BlockSpec((COMP_LEN, TB * NUM_TOKENS), lambda t: (0, t)),
            pl.BlockSpec((1, EMBED_DIM), lambda t: (0, 0)),
            pl.BlockSpec((1, EMBED_DIM), lambda t: (0, 0)),
            pl.BlockSpec((EMBED_DIM, COMP_DIM), lambda t: (0, 0)),
            pl.BlockSpec((COMP_LEN, COMP_DIM), lambda t: (0, 0)),
            pl.BlockSpec((COMP_LEN, COMP_DIM), lambda t: (0, 0)),
        ],
        out_specs=pl.BlockSpec((TB, COMP_LEN, COMP_DIM), lambda t: (t, 0, 0)),
        out_shape=jax.ShapeDtypeStruct((T, COMP_LEN, COMP_DIM), jnp.float32),
        interpret=interpret,
    )(x, w, g1, b1, pmat, g2, b2)


def kernel(x, ln1_scale, ln1_bias, ln2_scale, ln2_bias,
           node_token_pos, node_global_idx, stacked_indices):
    si = stacked_indices.astype(jnp.int32)
    nid = node_global_idx.astype(jnp.int32)
    pos = node_token_pos.astype(jnp.int32)

    w = _sc_weights(si, nid, pos)

    out = _tc_finish(x, w,
                     ln1_scale.reshape(1, EMBED_DIM),
                     ln1_bias.reshape(1, EMBED_DIM),
                     jnp.asarray(_PMAT),
                     ln2_scale.reshape(COMP_LEN, COMP_DIM),
                     ln2_bias.reshape(COMP_LEN, COMP_DIM))
    return out.reshape(T, 1, COMP_LEN * COMP_DIM)


# speedup vs baseline: 1.0638x; 1.0017x over previous
import functools

import numpy as np
import jax
import jax.numpy as jnp
from jax import lax
from jax.experimental import pallas as pl
from jax.experimental.pallas import tpu as pltpu
from jax.experimental.pallas import tpu_sc as plsc

T = 8
NUM_TOKENS = 2048
EMBED_DIM = 128
NUM_NODES = 50000
COMP_LEN = 8
COMP_DIM = 32
NN = 8192
MAX_LEN = 1024
POOL = EMBED_DIM // COMP_DIM
P_TOT = T * NUM_TOKENS

CH = 128
N_CH = NN // CH
H_CH = MAX_LEN // CH

NQ = 4
CPC = COMP_LEN // 2
NROW = 51200
ZCH = NROW // NQ
ACH = P_TOT // NQ
JCH = N_CH // NQ
HCH_T = H_CH // NQ

_ZEROS = np.zeros((ZCH,), np.float32)
_ONES = np.ones((CH,), np.float32)
_PMAT = np.repeat(np.eye(COMP_DIM, dtype=np.float32), POOL, axis=0) / POOL


def _sc_weights(si, nid, pos):
    mesh = plsc.VectorSubcoreMesh(core_axis_name="c", subcore_axis_name="s")

    @functools.partial(
        pl.kernel,
        out_type=jax.ShapeDtypeStruct((COMP_LEN, P_TOT), jnp.float32),
        mesh=mesh,
        scratch_types=[
            pltpu.VMEM_SHARED((CPC * NROW,), jnp.float32),
            pltpu.VMEM_SHARED((CPC * P_TOT,), jnp.float32),
            pltpu.VMEM((HCH_T, CH), jnp.int32),
            pltpu.VMEM((JCH * CH,), jnp.int32),
            pltpu.VMEM((JCH, CH), jnp.int32),
            pltpu.VMEM((JCH, CH), jnp.float32),
            pltpu.VMEM((CH,), jnp.float32),
            pltpu.VMEM((ZCH,), jnp.float32),
            pltpu.VMEM((ACH,), jnp.float32),
            pltpu.SemaphoreType.DMA,
            pltpu.SemaphoreType.DMA,
        ],
    )
    def k(si_hbm, nid_hbm, pos_hbm, zeros_hbm, ones_hbm, w_hbm,
          counts_s, acc_s, sif_v, nid_v, posf_v, wt_v, ones_v,
          zeros_v, stage_v, sem, sem2):
        cid = lax.axis_index("c")
        sid = lax.axis_index("s")
        lc = sid % NQ
        comp = cid + 2 * lc
        q = sid // NQ

        cnt_base = pl.multiple_of(lc * NROW + q * ZCH, 8)
        acc_base = pl.multiple_of(lc * P_TOT + q * ACH, 8)
        my_counts = counts_s.at[pl.ds(pl.multiple_of(lc * NROW, 8), NROW)]
        my_acc = acc_s.at[pl.ds(pl.multiple_of(lc * P_TOT, 8), P_TOT)]

        pend = [
            pltpu.async_copy(zeros_hbm, zeros_v, sem),
            pltpu.async_copy(ones_hbm, ones_v, sem),
            pltpu.async_copy(
                nid_hbm.at[pl.ds(q * (JCH * CH), JCH * CH)], nid_v, sem),
            pltpu.async_copy(si_hbm.at[comp].at[pl.ds(q * HCH_T, HCH_T)],
                             sif_v, sem),
            pltpu.async_copy(pos_hbm.at[pl.ds(q * JCH, JCH)], posf_v, sem),
        ]
        for d in pend:
            d.wait()
        pltpu.sync_copy(zeros_v, counts_s.at[pl.ds(cnt_base, ZCH)])
        plsc.subcore_barrier()

        pend = [
            pltpu.async_copy(ones_v, my_counts.at[sif_v.at[i]], sem, add=True)
            for i in range(HCH_T)
        ]
        pend.append(
            pltpu.async_copy(zeros_v.at[pl.ds(0, ACH)],
                             acc_s.at[pl.ds(acc_base, ACH)], sem))
        for d in pend:
            d.wait()
        plsc.subcore_barrier()

        HW = JCH // 2
        g1w = [
            pltpu.async_copy(my_counts.at[nid_v.at[pl.ds(i * CH, CH)]],
                             wt_v.at[i], sem)
            for i in range(HW)
        ]
        g2w = [
            pltpu.async_copy(my_counts.at[nid_v.at[pl.ds(i * CH, CH)]],
                             wt_v.at[i], sem2)
            for i in range(HW, JCH)
        ]
        for d in g1w:
            d.wait()
        s1w = [
            pltpu.async_copy(wt_v.at[i], my_acc.at[posf_v.at[i]], sem, add=True)
            for i in range(HW)
        ]
        for d in g2w:
            d.wait()
        s2w = [
            pltpu.async_copy(wt_v.at[i], my_acc.at[posf_v.at[i]], sem2,
                             add=True)
            for i in range(HW, JCH)
        ]
        for d in s1w:
            d.wait()
        for d in s2w:
            d.wait()
        plsc.subcore_barrier()

        pltpu.sync_copy(acc_s.at[pl.ds(acc_base, ACH)], stage_v)
        pltpu.sync_copy(stage_v, w_hbm.at[comp].at[pl.ds(q * ACH, ACH)])

    return k(si.reshape(COMP_LEN, H_CH, CH), nid, pos.reshape(N_CH, CH),
             jnp.asarray(_ZEROS), jnp.asarray(_ONES))


def _tc_finish(x, w, g1, b1, pmat, g2, b2, interpret=False):
    TB = 2

    def body(x_ref, w_ref, g1_ref, b1_ref, pm_ref, g2_ref, b2_ref, o_ref):
        for k in range(TB):
            xt = x_ref[k]
            mu = jnp.mean(xt, axis=-1, keepdims=True)
            var = jnp.mean(jnp.square(xt - mu), axis=-1, keepdims=True)
            xn = (xt - mu) * lax.rsqrt(var + 1e-5) * g1_ref[0] + b1_ref[0]
            acc = jnp.dot(w_ref[:, k * NUM_TOKENS:(k + 1) * NUM_TOKENS], xn,
                          preferred_element_type=jnp.float32)
            comp = jnp.dot(acc, pm_ref[...],
                           preferred_element_type=jnp.float32) * (1.0 / MAX_LEN)
            m2 = jnp.mean(comp)
            v2 = jnp.mean(jnp.square(comp - m2))
            o_ref[k] = ((comp - m2) * lax.rsqrt(v2 + 1e-5) * g2_ref[...]
                        + b2_ref[...])

    return pl.pallas_call(
        body,
        grid=(T // TB,),
        in_specs=[
            pl.BlockSpec((TB, NUM_TOKENS, EMBED_DIM), lambda t: (t, 0, 0)),
            pl.BlockSpec((COMP_LEN, TB * NUM_TOKENS), lambda t: (0, t)),
            pl.BlockSpec((1, EMBED_DIM), lambda t: (0, 0)),
            pl.BlockSpec((1, EMBED_DIM), lambda t: (0, 0)),
            pl.BlockSpec((EMBED_DIM, COMP_DIM), lambda t: (0, 0)),
            pl.BlockSpec((COMP_LEN, COMP_DIM), lambda t: (0, 0)),
            pl.BlockSpec((COMP_LEN, COMP_DIM), lambda t: (0, 0)),
        ],
        out_specs=pl.BlockSpec((TB, COMP_LEN, COMP_DIM), lambda t: (t, 0, 0)),
        out_shape=jax.ShapeDtypeStruct((T, COMP_LEN, COMP_DIM), jnp.float32),
        interpret=interpret,
    )(x, w, g1, b1, pmat, g2, b2)


def kernel(x, ln1_scale, ln1_bias, ln2_scale, ln2_bias,
           node_token_pos, node_global_idx, stacked_indices):
    si = stacked_indices.astype(jnp.int32)
    nid = node_global_idx.astype(jnp.int32)
    pos = node_token_pos.astype(jnp.int32)

    w = _sc_weights(si, nid, pos)

    out = _tc_finish(x, w,
                     ln1_scale.reshape(1, EMBED_DIM),
                     ln1_bias.reshape(1, EMBED_DIM),
                     jnp.asarray(_PMAT),
                     ln2_scale.reshape(COMP_LEN, COMP_DIM),
                     ln2_bias.reshape(COMP_LEN, COMP_DIM))
    return out.reshape(T, 1, COMP_LEN * COMP_DIM)
